# baseline (speedup 1.0000x reference)
import functools

import jax
import jax.numpy as jnp
from jax import lax
from jax.experimental import pallas as pl
from jax.experimental.pallas import tpu as pltpu
from jax.experimental.pallas import tpu_sc as plsc

_NC = 2
_NS = 16
_NW = _NC * _NS
_D = 64
_B = 4
_SEQ = 2048
_V = 100000
_TOT = _B * _SEQ
_FPW = _D // _NW
_LANES = 16
_GROUPS = _TOT // _LANES
_UNROLL = 4


def _emb_body(x_hbm, tokT_hbm, posT_hbm, out_hbm, idx_v, feat_v, pos_v, out_v, sem):
    c = lax.axis_index("c")
    s = lax.axis_index("s")
    wid = s * _NC + c
    for b in range(_B):
        pltpu.sync_copy(x_hbm.at[b], idx_v.at[pl.ds(b * _SEQ, _SEQ)])

    cps = []
    for j in range(_FPW):
        f = wid * _FPW + j
        pltpu.sync_copy(tokT_hbm.at[f], feat_v)
        pltpu.sync_copy(posT_hbm.at[f], pos_v)

        def body(g, carry):
            p0 = lax.rem(g * (_LANES * _UNROLL), _SEQ)
            offs = [g * (_LANES * _UNROLL) + u * _LANES for u in range(_UNROLL)]
            vecs = [idx_v[pl.ds(off, _LANES)] for off in offs]
            vals = [plsc.load_gather(feat_v, [vec]) for vec in vecs]
            pvs = [pos_v[pl.ds(p0 + u * _LANES, _LANES)] for u in range(_UNROLL)]
            for u in range(_UNROLL):
                out_v[j, pl.ds(offs[u], _LANES)] = vals[u] + pvs[u]
            return carry

        lax.fori_loop(0, _GROUPS // _UNROLL, body, 0)
        for b in range(_B):
            cps.append(
                pltpu.async_copy(
                    out_v.at[j, pl.ds(b * _SEQ, _SEQ)], out_hbm.at[b, f], sem
                )
            )
    for cp in cps:
        cp.wait()


@jax.jit
def kernel(x, token_table, pos_table):
    mesh = plsc.VectorSubcoreMesh(core_axis_name="c", subcore_axis_name="s")
    f = pl.kernel(
        _emb_body,
        out_type=jax.ShapeDtypeStruct((_B, _D, _SEQ), jnp.float32),
        mesh=mesh,
        scratch_types=[
            pltpu.VMEM((_TOT,), jnp.int32),
            pltpu.VMEM((_V,), jnp.float32),
            pltpu.VMEM((_SEQ,), jnp.float32),
            pltpu.VMEM((_FPW, _TOT), jnp.float32),
            pltpu.SemaphoreType.DMA,
        ],
        compiler_params=pltpu.CompilerParams(needs_layout_passes=False),
    )
    out = f(x, token_table.T, pos_table.T)
    return out.transpose(0, 2, 1)


# speedup vs baseline: 1.0897x; 1.0897x over previous
import functools

import jax
import jax.numpy as jnp
from jax import lax
from jax.experimental import pallas as pl
from jax.experimental.pallas import tpu as pltpu
from jax.experimental.pallas import tpu_sc as plsc

_NC = 2
_NS = 16
_NW = _NC * _NS
_D = 64
_B = 4
_SEQ = 2048
_V = 100000
_H0 = 51200
_H1A = 48768
_TAILW = 128
_TAIL0 = _V - _TAILW
_TOT = _B * _SEQ
_FPW = _D // _NW
_LANES = 16
_GROUPS = _TOT // _LANES
_UNROLL = 8


def _emb_body(x_hbm, tokT_hbm, posT_hbm, tailT_hbm, out_hbm,
              idx_v, feat_v, tail_v, pos_v, out_v, sem, xsem):
    c = lax.axis_index("c")
    s = lax.axis_index("s")
    wid = s * _NC + c
    lane_iota = lax.iota(jnp.int32, _LANES)
    stages = [(j, h) for j in range(_FPW) for h in range(2)]

    def half_dma(stage):
        j, h = stages[stage]
        row = tokT_hbm.at[wid * _FPW + j]
        if h == 0:
            return [
                pltpu.async_copy(row.at[pl.ds(0, _H0)], feat_v.at[pl.ds(0, _H0)], sem),
            ]
        return [
            pltpu.async_copy(row.at[pl.ds(_H0, _H1A)], feat_v.at[pl.ds(_H0, _H1A)], sem),
            pltpu.async_copy(tailT_hbm.at[wid * _FPW + j], tail_v, sem),
        ]

    cps = half_dma(0)
    for j in range(_FPW):
        pltpu.sync_copy(posT_hbm.at[wid * _FPW + j], pos_v.at[pl.ds(j * _SEQ, _SEQ)])
    for b in range(_B):
        pltpu.sync_copy(x_hbm.at[b], idx_v.at[pl.ds(b * _SEQ, _SEQ)])

    out_cps = []
    for stage, (j, h) in enumerate(stages):
        for cp in cps:
            cp.wait()
        if stage + 1 < len(stages):
            cps = half_dma(stage + 1)
        if h == 1:
            for t in range(_TAILW // _LANES):
                feat_v[pl.ds(_TAIL0 + t * _LANES, _LANES)] = tail_v[pl.ds(t * _LANES, _LANES)]

        def body(g, carry):
            p0 = lax.rem(g * (_LANES * _UNROLL), _SEQ)
            offs = [g * (_LANES * _UNROLL) + u * _LANES for u in range(_UNROLL)]
            vecs = [idx_v[pl.ds(off, _LANES)] for off in offs]
            if h == 0:
                vals = [plsc.load_gather(feat_v, [vec]) for vec in vecs]
            else:
                ms = [vec >= _H0 for vec in vecs]
                vals = [
                    plsc.load_gather(feat_v, [vec], mask=m)
                    for vec, m in zip(vecs, ms)
                ]
            pvs = [pos_v[pl.ds(j * _SEQ + p0 + u * _LANES, _LANES)] for u in range(_UNROLL)]
            if h == 0:
                for u in range(_UNROLL):
                    out_v[pl.ds(j * _TOT + offs[u], _LANES)] = vals[u] + pvs[u]
            else:
                for u in range(_UNROLL):
                    plsc.store_scatter(
                        out_v, [lane_iota + (j * _TOT + offs[u])], vals[u] + pvs[u], mask=ms[u]
                    )
            return carry

        lax.fori_loop(0, _GROUPS // _UNROLL, body, 0)

        if h == 1:
            f = wid * _FPW + j
            out_cps += [
                pltpu.async_copy(
                    out_v.at[pl.ds(j * _TOT + b * _SEQ, _SEQ)], out_hbm.at[b, f], xsem
                )
                for b in range(_B)
            ]
    for cp in out_cps:
        cp.wait()


@jax.jit
def kernel(x, token_table, pos_table):
    mesh = plsc.VectorSubcoreMesh(core_axis_name="c", subcore_axis_name="s")
    f = pl.kernel(
        _emb_body,
        out_type=jax.ShapeDtypeStruct((_B, _D, _SEQ), jnp.float32),
        mesh=mesh,
        scratch_types=[
            pltpu.VMEM((_TOT,), jnp.int32),
            pltpu.VMEM((_V,), jnp.float32),
            pltpu.VMEM((_TAILW,), jnp.float32),
            pltpu.VMEM((_FPW * _SEQ,), jnp.float32),
            pltpu.VMEM((_FPW * _TOT,), jnp.float32),
            pltpu.SemaphoreType.DMA,
            pltpu.SemaphoreType.DMA,
        ],
        compiler_params=pltpu.CompilerParams(needs_layout_passes=False),
    )
    tokT = token_table.T
    tailT = lax.slice(tokT, (0, _TAIL0), (_D, _V))
    out = f(x, tokT, pos_table.T, tailT)
    return out.transpose(0, 2, 1)


# speedup vs baseline: 1.0908x; 1.0011x over previous
import functools

import jax
import jax.numpy as jnp
from jax import lax
from jax.experimental import pallas as pl
from jax.experimental.pallas import tpu as pltpu
from jax.experimental.pallas import tpu_sc as plsc

_NC = 2
_NS = 16
_NW = _NC * _NS
_D = 64
_B = 4
_SEQ = 2048
_V = 100000
_H0 = 51200
_H1A = 48768
_TAILW = 128
_TAIL0 = _V - _TAILW
_TOT = _B * _SEQ
_FPW = _D // _NW
_LANES = 16
_GROUPS = _TOT // _LANES
_UNROLL = 8


def _emb_body(x_hbm, tokT_hbm, posT_hbm, tailT_hbm, out_hbm,
              idx_v, feat_v, tail_v, pos_v, out_v, sem, xsem):
    c = lax.axis_index("c")
    s = lax.axis_index("s")
    wid = s * _NC + c
    lane_iota = lax.iota(jnp.int32, _LANES)
    stages = [(j, h) for j in range(_FPW) for h in range(2)]

    def half_dma(stage):
        j, h = stages[stage]
        row = tokT_hbm.at[wid * _FPW + j]
        if h == 0:
            return [
                pltpu.async_copy(row.at[pl.ds(0, _H0)], feat_v.at[pl.ds(0, _H0)], sem),
            ]
        return [
            pltpu.async_copy(row.at[pl.ds(_H0, _H1A)], feat_v.at[pl.ds(_H0, _H1A)], sem),
            pltpu.async_copy(tailT_hbm.at[wid * _FPW + j], tail_v, sem),
        ]

    cps = half_dma(0)
    for j in range(_FPW):
        pltpu.sync_copy(posT_hbm.at[wid * _FPW + j], pos_v.at[pl.ds(j * _SEQ, _SEQ)])
    for b in range(_B):
        pltpu.sync_copy(x_hbm.at[b], idx_v.at[pl.ds(b * _SEQ, _SEQ)])

    out_cps = []
    for stage, (j, h) in enumerate(stages):
        for cp in cps:
            cp.wait()
        if stage + 1 < len(stages):
            cps = half_dma(stage + 1)
        if h == 1:
            for t in range(_TAILW // _LANES):
                feat_v[pl.ds(_TAIL0 + t * _LANES, _LANES)] = tail_v[pl.ds(t * _LANES, _LANES)]

        def body(g, carry):
            p0 = lax.rem(g * (_LANES * _UNROLL), _SEQ)
            offs = [g * (_LANES * _UNROLL) + u * _LANES for u in range(_UNROLL)]
            vecs = [idx_v[pl.ds(off, _LANES)] for off in offs]
            if h == 0:
                ms = [vec < _H0 for vec in vecs]
            else:
                ms = [vec >= _H0 for vec in vecs]
            vals = [
                plsc.load_gather(feat_v, [vec], mask=m)
                for vec, m in zip(vecs, ms)
            ]
            pvs = [pos_v[pl.ds(j * _SEQ + p0 + u * _LANES, _LANES)] for u in range(_UNROLL)]
            if h == 0:
                for u in range(_UNROLL):
                    out_v[pl.ds(j * _TOT + offs[u], _LANES)] = vals[u] + pvs[u]
            else:
                for u in range(_UNROLL):
                    plsc.store_scatter(
                        out_v, [lane_iota + (j * _TOT + offs[u])], vals[u] + pvs[u], mask=ms[u]
                    )
            return carry

        lax.fori_loop(0, _GROUPS // _UNROLL, body, 0)

        if h == 1:
            f = wid * _FPW + j
            out_cps += [
                pltpu.async_copy(
                    out_v.at[pl.ds(j * _TOT + b * _SEQ, _SEQ)], out_hbm.at[b, f], xsem
                )
                for b in range(_B)
            ]
    for cp in out_cps:
        cp.wait()


@jax.jit
def kernel(x, token_table, pos_table):
    mesh = plsc.VectorSubcoreMesh(core_axis_name="c", subcore_axis_name="s")
    f = pl.kernel(
        _emb_body,
        out_type=jax.ShapeDtypeStruct((_B, _D, _SEQ), jnp.float32),
        mesh=mesh,
        scratch_types=[
            pltpu.VMEM((_TOT,), jnp.int32),
            pltpu.VMEM((_V,), jnp.float32),
            pltpu.VMEM((_TAILW,), jnp.float32),
            pltpu.VMEM((_FPW * _SEQ,), jnp.float32),
            pltpu.VMEM((_FPW * _TOT,), jnp.float32),
            pltpu.SemaphoreType.DMA,
            pltpu.SemaphoreType.DMA,
        ],
        compiler_params=pltpu.CompilerParams(needs_layout_passes=False),
    )
    tokT = token_table.T
    tailT = lax.slice(tokT, (0, _TAIL0), (_D, _V))
    out = f(x, tokT, pos_table.T, tailT)
    return out.transpose(0, 2, 1)
